# baseline (speedup 1.0000x reference)
import functools

import jax
import jax.numpy as jnp
from jax import lax
from jax.experimental import pallas as pl
from jax.experimental.pallas import tpu as pltpu
from jax.experimental.pallas import tpu_sc as plsc

N_T = 5000
N_P = 5000
N = 10000
E = 320000
D = 128
R = 5

NC = 2
NS = 16
NW = NC * NS
L = 16
EPW = E // NW
CH = 80
NCHUNK = EPW // CH
NR = N * R
NRP = 50176
DPT = NRP // NS
RPT = N // NS
BR = 1000


def _ln(y, g, b):
    m = jnp.mean(y, axis=-1, keepdims=True)
    v = jnp.mean((y - m) ** 2, axis=-1, keepdims=True)
    return (y - m) / jnp.sqrt(v + 1e-5) * g + b


def _pre_body(x_ref, w_ref, b_ref, g_ref, be_ref, wall_ref, o_ref, h_ref):
    y = jnp.dot(x_ref[0, 0], w_ref[0], preferred_element_type=jnp.float32)
    y = _ln(y + b_ref[0], g_ref[0], be_ref[0])
    y = jnp.maximum(y, 0.0)
    o_ref[...] = y
    for r in range(R + 1):
        h_ref[r] = jnp.dot(y, wall_ref[r], preferred_element_type=jnp.float32)


def _pre(tf, pf, Wt, bt, gt, bet, Wp, bp, gp, bep, wall1):
    nb = N_T // BR
    xs = jnp.stack([tf, pf]).reshape(2, nb, BR, D)
    ws = jnp.stack([Wt, Wp])
    bs = jnp.stack([bt, bp]).reshape(2, 1, D)
    gs = jnp.stack([gt, gp]).reshape(2, 1, D)
    bes = jnp.stack([bet, bep]).reshape(2, 1, D)
    return pl.pallas_call(
        _pre_body,
        grid=(N // BR,),
        in_specs=[
            pl.BlockSpec((1, 1, BR, D), lambda i: (i // nb, i % nb, 0, 0)),
            pl.BlockSpec((1, D, D), lambda i: (i // nb, 0, 0)),
            pl.BlockSpec((1, 1, D), lambda i: (i // nb, 0, 0)),
            pl.BlockSpec((1, 1, D), lambda i: (i // nb, 0, 0)),
            pl.BlockSpec((1, 1, D), lambda i: (i // nb, 0, 0)),
            pl.BlockSpec((R + 1, D, D), lambda i: (0, 0, 0)),
        ],
        out_specs=[
            pl.BlockSpec((BR, D), lambda i: (i, 0)),
            pl.BlockSpec((R + 1, BR, D), lambda i: (0, i, 0)),
        ],
        out_shape=[
            jax.ShapeDtypeStruct((N, D), jnp.float32),
            jax.ShapeDtypeStruct((R + 1, N, D), jnp.float32),
        ],
    )(xs, ws, bs, gs, bes, wall1)


def _post_last_body(part_ref, h_ref, bc_ref, g_ref, be_ref, res_ref, o_ref):
    y = part_ref[0] + part_ref[1] + h_ref[0] + bc_ref[...]
    y = _ln(y, g_ref[...], be_ref[...])
    o_ref[...] = y + res_ref[...]


def _post_next_body(part_ref, h_ref, bc_ref, g_ref, be_ref, res_ref, wall_ref,
                    o_ref, hn_ref):
    y = part_ref[0] + part_ref[1] + h_ref[0] + bc_ref[...]
    y = _ln(y, g_ref[...], be_ref[...])
    y = jnp.maximum(y, 0.0) + res_ref[...]
    o_ref[...] = y
    for r in range(R + 1):
        hn_ref[r] = jnp.dot(y, wall_ref[r], preferred_element_type=jnp.float32)


def _post(part, h6, bc, g, be, res, wall_next):
    base_specs = [
        pl.BlockSpec((NC, BR, D), lambda i: (0, i, 0)),
        pl.BlockSpec((1, BR, D), lambda i: (R, i, 0)),
        pl.BlockSpec((1, D), lambda i: (0, 0)),
        pl.BlockSpec((1, D), lambda i: (0, 0)),
        pl.BlockSpec((1, D), lambda i: (0, 0)),
        pl.BlockSpec((BR, D), lambda i: (i, 0)),
    ]
    args = (part, h6, bc.reshape(1, D), g.reshape(1, D), be.reshape(1, D), res)
    if wall_next is None:
        return pl.pallas_call(
            _post_last_body,
            grid=(N // BR,),
            in_specs=base_specs,
            out_specs=pl.BlockSpec((BR, D), lambda i: (i, 0)),
            out_shape=jax.ShapeDtypeStruct((N, D), jnp.float32),
        )(*args)
    return pl.pallas_call(
        _post_next_body,
        grid=(N // BR,),
        in_specs=base_specs + [pl.BlockSpec((R + 1, D, D), lambda i: (0, 0, 0))],
        out_specs=[
            pl.BlockSpec((BR, D), lambda i: (i, 0)),
            pl.BlockSpec((R + 1, BR, D), lambda i: (0, i, 0)),
        ],
        out_shape=[
            jax.ShapeDtypeStruct((N, D), jnp.float32),
            jax.ShapeDtypeStruct((R + 1, N, D), jnp.float32),
        ],
    )(*args, wall_next)


def _t_body(d_ref, o_ref):
    o_ref[...] = 1.0 / jnp.maximum(d_ref[0] + d_ref[1], 1.0)


def _t_table(deg2):
    out = pl.pallas_call(
        _t_body,
        in_specs=[pl.BlockSpec((NC, NRP // D, D), lambda: (0, 0, 0))],
        out_specs=pl.BlockSpec((NRP // D, D), lambda: (0, 0)),
        out_shape=jax.ShapeDtypeStruct((NRP // D, D), jnp.float32),
    )(deg2.reshape(NC, NRP // D, D))
    return out.reshape(NRP)


def _sc_mesh():
    return plsc.VectorSubcoreMesh(core_axis_name="c", subcore_axis_name="s")


def _deg_sc(dst, et):
    @functools.partial(
        pl.kernel,
        out_type=jax.ShapeDtypeStruct((NC * NRP,), jnp.float32),
        mesh=_sc_mesh(),
        scratch_types=[
            pltpu.VMEM((EPW,), jnp.int32),
            pltpu.VMEM((EPW,), jnp.int32),
            pltpu.VMEM((CH,), jnp.int32),
            pltpu.VMEM((CH,), jnp.float32),
            pltpu.VMEM((DPT,), jnp.float32),
            pltpu.VMEM_SHARED((NRP,), jnp.float32),
        ],
    )
    def k(dst_hbm, et_hbm, out_hbm, dst_all, et_all, key_v, ones_v, zb_v, acc):
        c = lax.axis_index("c")
        s = lax.axis_index("s")
        w = c * NS + s
        one16 = jnp.ones((L,), jnp.float32)
        z16 = jnp.zeros((L,), jnp.float32)

        def fill(i, _):
            ones_v[pl.ds(i * L, L)] = one16
            return 0

        lax.fori_loop(0, CH // L, fill, 0)

        def fillz(i, _):
            zb_v[pl.ds(i * L, L)] = z16
            return 0

        lax.fori_loop(0, DPT // L, fillz, 0)
        pltpu.sync_copy(zb_v, acc.at[pl.ds(s * DPT, DPT)])
        plsc.subcore_barrier()

        pltpu.sync_copy(dst_hbm.at[pl.ds(w * EPW, EPW)], dst_all)
        pltpu.sync_copy(et_hbm.at[pl.ds(w * EPW, EPW)], et_all)

        def chunk(cix, _):
            base = cix * CH
            for j in range(CH // L):
                pos = base + j * L
                d16 = dst_all[pl.ds(pos, L)]
                e16 = et_all[pl.ds(pos, L)]
                key_v[pl.ds(j * L, L)] = d16 * R + e16
            pltpu.sync_copy(ones_v, acc.at[key_v], add=True)
            return 0

        lax.fori_loop(0, NCHUNK, chunk, 0)
        plsc.subcore_barrier()
        pltpu.sync_copy(acc.at[pl.ds(s * DPT, DPT)], zb_v)
        pltpu.sync_copy(zb_v, out_hbm.at[pl.ds(c * NRP + s * DPT, DPT)])

    return k(dst, et).reshape(NC, NRP)


def _prep_sc(src, dst, et, tnorm):
    @functools.partial(
        pl.kernel,
        out_type=(
            jax.ShapeDtypeStruct((E,), jnp.float32),
            jax.ShapeDtypeStruct((E,), jnp.int32),
        ),
        mesh=_sc_mesh(),
        scratch_types=[
            pltpu.VMEM((EPW,), jnp.int32),
            pltpu.VMEM((EPW,), jnp.int32),
            pltpu.VMEM((EPW,), jnp.int32),
            pltpu.VMEM((EPW,), jnp.float32),
            pltpu.VMEM((EPW,), jnp.int32),
            pltpu.VMEM((CH,), jnp.int32),
            pltpu.VMEM((CH,), jnp.int32),
            pltpu.SemaphoreType.DMA,
            pltpu.SemaphoreType.DMA,
        ],
    )
    def k(src_hbm, dst_hbm, et_hbm, t_hbm, nout_hbm, gout_hbm,
          src_all, dst_all, et_all, norm_slab, gidx_slab, key_a, key_b,
          sem_a, sem_b):
        c = lax.axis_index("c")
        s = lax.axis_index("s")
        w = c * NS + s
        pltpu.sync_copy(src_hbm.at[pl.ds(w * EPW, EPW)], src_all)
        pltpu.sync_copy(dst_hbm.at[pl.ds(w * EPW, EPW)], dst_all)
        pltpu.sync_copy(et_hbm.at[pl.ds(w * EPW, EPW)], et_all)
        keys = (key_a, key_b)
        sems = (sem_a, sem_b)

        def gfill(i, _):
            pos = i * L
            gidx_slab[pl.ds(pos, L)] = (
                et_all[pl.ds(pos, L)] * N + src_all[pl.ds(pos, L)]
            )
            return 0

        lax.fori_loop(0, EPW // L, gfill, 0)
        pltpu.sync_copy(gidx_slab, gout_hbm.at[pl.ds(w * EPW, EPW)])

        def fill_keys(cix, kv):
            base = cix * CH
            for j in range(CH // L):
                pos = base + j * L
                kv[pl.ds(j * L, L)] = (
                    dst_all[pl.ds(pos, L)] * R + et_all[pl.ds(pos, L)]
                )

        def fire(cix, b):
            pltpu.async_copy(
                t_hbm.at[keys[b]], norm_slab.at[pl.ds(cix * CH, CH)], sems[b]
            )

        def drain(cix, b):
            pltpu.make_async_copy(
                t_hbm.at[keys[b]], norm_slab.at[pl.ds(cix * CH, CH)], sems[b]
            ).wait()

        fill_keys(0, key_a)
        fire(0, 0)

        def body(i, _):
            fill_keys(2 * i + 1, key_b)
            fire(2 * i + 1, 1)
            drain(2 * i, 0)

            @pl.when(i < NCHUNK // 2 - 1)
            def _():
                fill_keys(2 * i + 2, key_a)
                fire(2 * i + 2, 0)

            drain(2 * i + 1, 1)
            return 0

        lax.fori_loop(0, NCHUNK // 2, body, 0)
        fill_keys(NCHUNK - 1, key_a)
        fire(NCHUNK - 1, 0)
        drain(NCHUNK - 1, 0)
        pltpu.sync_copy(norm_slab, nout_hbm.at[pl.ds(w * EPW, EPW)])

    return k(src, dst, et, tnorm)


NB = 3
SUP = 25
SUPCH = SUP * CH


def _agg_sc(h5, gidx, dst, norm_all):
    @functools.partial(
        pl.kernel,
        out_type=jax.ShapeDtypeStruct((NC, N, D), jnp.float32),
        mesh=_sc_mesh(),
        scratch_types=[
            pltpu.VMEM((SUPCH,), jnp.int32),
            pltpu.VMEM((SUPCH,), jnp.int32),
            pltpu.VMEM((EPW,), jnp.float32),
            [pltpu.VMEM((CH,), jnp.int32)] * NB,
            [pltpu.VMEM((CH,), jnp.int32)] * NB,
            [pltpu.VMEM((CH, D), jnp.float32)] * NB,
            [pltpu.SemaphoreType.DMA] * NB,
            [pltpu.SemaphoreType.DMA] * NB,
            pltpu.VMEM_SHARED((N, D), jnp.float32),
        ],
    )
    def k(h_hbm, gidx_hbm, dst_hbm, n_hbm, out_hbm,
          gidx_sb, dst_sb, norm_all_v, idx_b, dsti_b, rows_b,
          sem_g, sem_s, acc):
        c = lax.axis_index("c")
        s = lax.axis_index("s")
        w = c * NS + s
        z16 = jnp.zeros((L,), jnp.float32)
        rows_v = rows_b[0]

        def zr(i, _):
            for j in range(D // L):
                rows_v[i, pl.ds(j * L, L)] = z16
            return 0

        lax.fori_loop(0, CH, zr, 0)
        row0 = pl.multiple_of(s * 624, 8)

        @pl.when(s < NS - 1)
        def _zero_main():
            for kk in range(7):
                pltpu.sync_copy(rows_v, acc.at[pl.ds(row0 + kk * CH, CH)])
            pltpu.sync_copy(
                rows_v.at[pl.ds(0, 64)], acc.at[pl.ds(row0 + 7 * CH, 64)]
            )

        @pl.when(s == NS - 1)
        def _zero_last():
            for kk in range(8):
                pltpu.sync_copy(rows_v, acc.at[pl.ds(9360 + kk * CH, CH)])

        plsc.subcore_barrier()

        pltpu.sync_copy(n_hbm.at[pl.ds(w * EPW, EPW)], norm_all_v)

        def refill(cix):
            soff = w * EPW + cix * CH
            pltpu.sync_copy(gidx_hbm.at[pl.ds(soff, SUPCH)], gidx_sb)
            pltpu.sync_copy(dst_hbm.at[pl.ds(soff, SUPCH)], dst_sb)

        def fill_idx(cix, b):
            base = lax.rem(cix, SUP) * CH
            for j in range(CH // L):
                pos = base + j * L
                idx_b[b][pl.ds(j * L, L)] = gidx_sb[pl.ds(pos, L)]
                dsti_b[b][pl.ds(j * L, L)] = dst_sb[pl.ds(pos, L)]

        def fire_gather(b):
            pltpu.async_copy(h_hbm.at[idx_b[b]], rows_b[b], sem_g[b])

        def wait_gather(b):
            pltpu.make_async_copy(h_hbm.at[idx_b[b]], rows_b[b], sem_g[b]).wait()

        def fire_scatter(b):
            pltpu.async_copy(rows_b[b], acc.at[dsti_b[b]], sem_s[b], add=True)

        def wait_scatter(b):
            pltpu.make_async_copy(rows_b[b], acc.at[dsti_b[b]], sem_s[b]).wait()

        def scale(cix, b):
            rv = rows_b[b]
            for g in range(CH // L):
                norm16 = norm_all_v[pl.ds(cix * CH + g * L, L)]
                for e in range(L):
                    nb = jnp.take(norm16, jnp.full((L,), e, jnp.int32))
                    for j in range(D // L):
                        rv[g * L + e, pl.ds(j * L, L)] = (
                            rv[g * L + e, pl.ds(j * L, L)] * nb
                        )

        refill(0)
        for b in range(NB):
            fill_idx(b, b)
            fire_gather(b)

        def body(i, _):
            for b in range(NB):
                cix = i * NB + b
                bp = (b - 1) % NB

                @pl.when(cix < NCHUNK)
                def _(cix=cix, b=b, bp=bp):
                    wait_gather(b)
                    scale(cix, b)
                    fire_scatter(b)
                    p = cix - 1

                    @pl.when((cix >= 1) & (p + NB < NCHUNK))
                    def _():
                        wait_scatter(bp)

                        @pl.when(lax.rem(p + NB, SUP) == 0)
                        def _():
                            refill(p + NB)

                        fill_idx(p + NB, bp)
                        fire_gather(bp)

            return 0

        lax.fori_loop(0, (NCHUNK + NB - 1) // NB, body, 0)
        for b in range(NB):
            wait_scatter(b)
        plsc.subcore_barrier()

        @pl.when(s < NS - 1)
        def _out_main():
            for kk in range(7):
                pltpu.sync_copy(acc.at[pl.ds(row0 + kk * CH, CH)], rows_v)
                pltpu.sync_copy(rows_v, out_hbm.at[c, pl.ds(row0 + kk * CH, CH)])
            pltpu.sync_copy(
                acc.at[pl.ds(row0 + 7 * CH, 64)], rows_v.at[pl.ds(0, 64)]
            )
            pltpu.sync_copy(
                rows_v.at[pl.ds(0, 64)], out_hbm.at[c, pl.ds(row0 + 7 * CH, 64)]
            )

        @pl.when(s == NS - 1)
        def _out_last():
            for kk in range(8):
                pltpu.sync_copy(acc.at[pl.ds(9360 + kk * CH, CH)], rows_v)
                pltpu.sync_copy(rows_v, out_hbm.at[c, pl.ds(9360 + kk * CH, CH)])

    return k(h5, gidx, dst, norm_all)


def kernel(thesis_features, professor_features, edge_index, edge_type,
           Wt, bt, gt, bet, Wp, bp, gp, bep,
           Wr1, Wroot1, bc1, g1, be1,
           Wr2, Wroot2, bc2, g2, be2,
           Wr3, Wroot3, bc3, g3, be3):
    src = edge_index[0].astype(jnp.int32)
    dst = edge_index[1].astype(jnp.int32)
    et = edge_type.astype(jnp.int32)

    wall2 = jnp.concatenate([Wr2, Wroot2[None]], axis=0)
    wall3 = jnp.concatenate([Wr3, Wroot3[None]], axis=0)

    x0, h6 = _pre(thesis_features, professor_features,
                  Wt, bt, gt, bet, Wp, bp, gp, bep,
                  jnp.concatenate([Wr1, Wroot1[None]], axis=0))
    deg2 = _deg_sc(dst, et)
    tnorm = _t_table(deg2)
    norm_all, gidx = _prep_sc(src, dst, et, tnorm)

    x = x0
    for bc, g, be, wall_next in (
        (bc1, g1, be1, wall2),
        (bc2, g2, be2, wall3),
        (bc3, g3, be3, None),
    ):
        part = _agg_sc(h6.reshape((R + 1) * N, D), gidx, dst, norm_all)
        if wall_next is None:
            x = _post(part, h6, bc, g, be, x, None)
        else:
            x, h6 = _post(part, h6, bc, g, be, x, wall_next)

    return (x0, x)


# speedup vs baseline: 1.2840x; 1.2840x over previous
import functools

import jax
import jax.numpy as jnp
from jax import lax
from jax.experimental import pallas as pl
from jax.experimental.pallas import tpu as pltpu
from jax.experimental.pallas import tpu_sc as plsc

N_T = 5000
N_P = 5000
N = 10000
E = 320000
D = 128
R = 5

NC = 2
NS = 16
NW = NC * NS
L = 16
EPW = E // NW
CH = 80
NCHUNK = EPW // CH
NR = N * R
NRP = 50176
DPT = NRP // NS
RPT = N // NS
BR = 1000


def _ln(y, g, b):
    m = jnp.mean(y, axis=-1, keepdims=True)
    v = jnp.mean((y - m) ** 2, axis=-1, keepdims=True)
    return (y - m) / jnp.sqrt(v + 1e-5) * g + b


def _pre_body(x_ref, w_ref, b_ref, g_ref, be_ref, wall_ref, o_ref, h_ref):
    y = jnp.dot(x_ref[0, 0], w_ref[0], preferred_element_type=jnp.float32)
    y = _ln(y + b_ref[0], g_ref[0], be_ref[0])
    y = jnp.maximum(y, 0.0)
    o_ref[...] = y
    for r in range(R + 1):
        h_ref[r] = jnp.dot(y, wall_ref[r], preferred_element_type=jnp.float32)


def _pre(tf, pf, Wt, bt, gt, bet, Wp, bp, gp, bep, wall1):
    nb = N_T // BR
    xs = jnp.stack([tf, pf]).reshape(2, nb, BR, D)
    ws = jnp.stack([Wt, Wp])
    bs = jnp.stack([bt, bp]).reshape(2, 1, D)
    gs = jnp.stack([gt, gp]).reshape(2, 1, D)
    bes = jnp.stack([bet, bep]).reshape(2, 1, D)
    return pl.pallas_call(
        _pre_body,
        grid=(N // BR,),
        in_specs=[
            pl.BlockSpec((1, 1, BR, D), lambda i: (i // nb, i % nb, 0, 0)),
            pl.BlockSpec((1, D, D), lambda i: (i // nb, 0, 0)),
            pl.BlockSpec((1, 1, D), lambda i: (i // nb, 0, 0)),
            pl.BlockSpec((1, 1, D), lambda i: (i // nb, 0, 0)),
            pl.BlockSpec((1, 1, D), lambda i: (i // nb, 0, 0)),
            pl.BlockSpec((R + 1, D, D), lambda i: (0, 0, 0)),
        ],
        out_specs=[
            pl.BlockSpec((BR, D), lambda i: (i, 0)),
            pl.BlockSpec((R + 1, BR, D), lambda i: (0, i, 0)),
        ],
        out_shape=[
            jax.ShapeDtypeStruct((N, D), jnp.float32),
            jax.ShapeDtypeStruct((R + 1, N, D), jnp.float32),
        ],
    )(xs, ws, bs, gs, bes, wall1)


def _post_last_body(part_ref, h_ref, bc_ref, g_ref, be_ref, res_ref, o_ref):
    y = part_ref[0] + part_ref[1] + h_ref[0] + bc_ref[...]
    y = _ln(y, g_ref[...], be_ref[...])
    o_ref[...] = y + res_ref[...]


def _post_next_body(part_ref, h_ref, bc_ref, g_ref, be_ref, res_ref, wall_ref,
                    o_ref, hn_ref):
    y = part_ref[0] + part_ref[1] + h_ref[0] + bc_ref[...]
    y = _ln(y, g_ref[...], be_ref[...])
    y = jnp.maximum(y, 0.0) + res_ref[...]
    o_ref[...] = y
    for r in range(R + 1):
        hn_ref[r] = jnp.dot(y, wall_ref[r], preferred_element_type=jnp.float32)


def _post(part, h6, bc, g, be, res, wall_next):
    base_specs = [
        pl.BlockSpec((NC, BR, D), lambda i: (0, i, 0)),
        pl.BlockSpec((1, BR, D), lambda i: (R, i, 0)),
        pl.BlockSpec((1, D), lambda i: (0, 0)),
        pl.BlockSpec((1, D), lambda i: (0, 0)),
        pl.BlockSpec((1, D), lambda i: (0, 0)),
        pl.BlockSpec((BR, D), lambda i: (i, 0)),
    ]
    args = (part, h6, bc.reshape(1, D), g.reshape(1, D), be.reshape(1, D), res)
    if wall_next is None:
        return pl.pallas_call(
            _post_last_body,
            grid=(N // BR,),
            in_specs=base_specs,
            out_specs=pl.BlockSpec((BR, D), lambda i: (i, 0)),
            out_shape=jax.ShapeDtypeStruct((N, D), jnp.float32),
        )(*args)
    return pl.pallas_call(
        _post_next_body,
        grid=(N // BR,),
        in_specs=base_specs + [pl.BlockSpec((R + 1, D, D), lambda i: (0, 0, 0))],
        out_specs=[
            pl.BlockSpec((BR, D), lambda i: (i, 0)),
            pl.BlockSpec((R + 1, BR, D), lambda i: (0, i, 0)),
        ],
        out_shape=[
            jax.ShapeDtypeStruct((N, D), jnp.float32),
            jax.ShapeDtypeStruct((R + 1, N, D), jnp.float32),
        ],
    )(*args, wall_next)


def _t_body(d_ref, o_ref):
    o_ref[...] = 1.0 / jnp.maximum(d_ref[0] + d_ref[1], 1.0)


def _t_table(deg2):
    out = pl.pallas_call(
        _t_body,
        in_specs=[pl.BlockSpec((NC, NRP // D, D), lambda: (0, 0, 0))],
        out_specs=pl.BlockSpec((NRP // D, D), lambda: (0, 0)),
        out_shape=jax.ShapeDtypeStruct((NRP // D, D), jnp.float32),
    )(deg2.reshape(NC, NRP // D, D))
    return out.reshape(NRP)


def _sc_mesh():
    return plsc.VectorSubcoreMesh(core_axis_name="c", subcore_axis_name="s")


def _deg_sc(dst, et):
    @functools.partial(
        pl.kernel,
        out_type=jax.ShapeDtypeStruct((NC * NRP,), jnp.float32),
        mesh=_sc_mesh(),
        scratch_types=[
            pltpu.VMEM((EPW,), jnp.int32),
            pltpu.VMEM((EPW,), jnp.int32),
            pltpu.VMEM((CH,), jnp.int32),
            pltpu.VMEM((CH,), jnp.float32),
            pltpu.VMEM((DPT,), jnp.float32),
            pltpu.VMEM_SHARED((NRP,), jnp.float32),
        ],
    )
    def k(dst_hbm, et_hbm, out_hbm, dst_all, et_all, key_v, ones_v, zb_v, acc):
        c = lax.axis_index("c")
        s = lax.axis_index("s")
        w = c * NS + s
        one16 = jnp.ones((L,), jnp.float32)
        z16 = jnp.zeros((L,), jnp.float32)

        def fill(i, _):
            ones_v[pl.ds(i * L, L)] = one16
            return 0

        lax.fori_loop(0, CH // L, fill, 0)

        def fillz(i, _):
            zb_v[pl.ds(i * L, L)] = z16
            return 0

        lax.fori_loop(0, DPT // L, fillz, 0)
        pltpu.sync_copy(zb_v, acc.at[pl.ds(s * DPT, DPT)])
        plsc.subcore_barrier()

        pltpu.sync_copy(dst_hbm.at[pl.ds(w * EPW, EPW)], dst_all)
        pltpu.sync_copy(et_hbm.at[pl.ds(w * EPW, EPW)], et_all)

        def chunk(cix, _):
            base = cix * CH
            for j in range(CH // L):
                pos = base + j * L
                d16 = dst_all[pl.ds(pos, L)]
                e16 = et_all[pl.ds(pos, L)]
                key_v[pl.ds(j * L, L)] = d16 * R + e16
            pltpu.sync_copy(ones_v, acc.at[key_v], add=True)
            return 0

        lax.fori_loop(0, NCHUNK, chunk, 0)
        plsc.subcore_barrier()
        pltpu.sync_copy(acc.at[pl.ds(s * DPT, DPT)], zb_v)
        pltpu.sync_copy(zb_v, out_hbm.at[pl.ds(c * NRP + s * DPT, DPT)])

    return k(dst, et).reshape(NC, NRP)


def _prep_sc(src, dst, et, tnorm):
    @functools.partial(
        pl.kernel,
        out_type=(
            jax.ShapeDtypeStruct((E,), jnp.float32),
            jax.ShapeDtypeStruct((E,), jnp.int32),
        ),
        mesh=_sc_mesh(),
        scratch_types=[
            pltpu.VMEM((EPW,), jnp.int32),
            pltpu.VMEM((EPW,), jnp.int32),
            pltpu.VMEM((EPW,), jnp.int32),
            pltpu.VMEM((EPW,), jnp.float32),
            pltpu.VMEM((EPW,), jnp.int32),
            pltpu.VMEM((CH,), jnp.int32),
            pltpu.VMEM((CH,), jnp.int32),
            pltpu.SemaphoreType.DMA,
            pltpu.SemaphoreType.DMA,
        ],
    )
    def k(src_hbm, dst_hbm, et_hbm, t_hbm, nout_hbm, gout_hbm,
          src_all, dst_all, et_all, norm_slab, gidx_slab, key_a, key_b,
          sem_a, sem_b):
        c = lax.axis_index("c")
        s = lax.axis_index("s")
        w = c * NS + s
        pltpu.sync_copy(src_hbm.at[pl.ds(w * EPW, EPW)], src_all)
        pltpu.sync_copy(dst_hbm.at[pl.ds(w * EPW, EPW)], dst_all)
        pltpu.sync_copy(et_hbm.at[pl.ds(w * EPW, EPW)], et_all)
        keys = (key_a, key_b)
        sems = (sem_a, sem_b)

        def gfill(i, _):
            pos = i * L
            gidx_slab[pl.ds(pos, L)] = (
                et_all[pl.ds(pos, L)] * N + src_all[pl.ds(pos, L)]
            )
            return 0

        lax.fori_loop(0, EPW // L, gfill, 0)
        pltpu.sync_copy(gidx_slab, gout_hbm.at[pl.ds(w * EPW, EPW)])

        def fill_keys(cix, kv):
            base = cix * CH
            for j in range(CH // L):
                pos = base + j * L
                kv[pl.ds(j * L, L)] = (
                    dst_all[pl.ds(pos, L)] * R + et_all[pl.ds(pos, L)]
                )

        def fire(cix, b):
            pltpu.async_copy(
                t_hbm.at[keys[b]], norm_slab.at[pl.ds(cix * CH, CH)], sems[b]
            )

        def drain(cix, b):
            pltpu.make_async_copy(
                t_hbm.at[keys[b]], norm_slab.at[pl.ds(cix * CH, CH)], sems[b]
            ).wait()

        fill_keys(0, key_a)
        fire(0, 0)

        def body(i, _):
            fill_keys(2 * i + 1, key_b)
            fire(2 * i + 1, 1)
            drain(2 * i, 0)

            @pl.when(i < NCHUNK // 2 - 1)
            def _():
                fill_keys(2 * i + 2, key_a)
                fire(2 * i + 2, 0)

            drain(2 * i + 1, 1)
            return 0

        lax.fori_loop(0, NCHUNK // 2, body, 0)
        fill_keys(NCHUNK - 1, key_a)
        fire(NCHUNK - 1, 0)
        drain(NCHUNK - 1, 0)
        pltpu.sync_copy(norm_slab, nout_hbm.at[pl.ds(w * EPW, EPW)])

    return k(src, dst, et, tnorm)


NB = 3
SUP = 25
SUPCH = SUP * CH


def _agg_sc(h5, gidx, dst, norm_all):
    @functools.partial(
        pl.kernel,
        out_type=jax.ShapeDtypeStruct((NC, N, D), jnp.float32),
        mesh=_sc_mesh(),
        scratch_types=[
            pltpu.VMEM((SUPCH,), jnp.int32),
            pltpu.VMEM((SUPCH,), jnp.int32),
            pltpu.VMEM((EPW,), jnp.float32),
            [pltpu.VMEM((CH,), jnp.int32)] * NB,
            [pltpu.VMEM((CH,), jnp.int32)] * NB,
            [pltpu.VMEM((CH, D), jnp.float32)] * NB,
            [pltpu.SemaphoreType.DMA] * NB,
            [pltpu.SemaphoreType.DMA] * NB,
            pltpu.VMEM_SHARED((N, D), jnp.float32),
        ],
    )
    def k(h_hbm, gidx_hbm, dst_hbm, n_hbm, out_hbm,
          gidx_sb, dst_sb, norm_all_v, idx_b, dsti_b, rows_b,
          sem_g, sem_s, acc):
        c = lax.axis_index("c")
        s = lax.axis_index("s")
        w = c * NS + s
        z16 = jnp.zeros((L,), jnp.float32)
        rows_v = rows_b[0]

        def zr(i, _):
            for j in range(D // L):
                rows_v[i, pl.ds(j * L, L)] = z16
            return 0

        lax.fori_loop(0, CH, zr, 0)
        row0 = pl.multiple_of(s * 624, 8)

        @pl.when(s < NS - 1)
        def _zero_main():
            for kk in range(7):
                pltpu.sync_copy(rows_v, acc.at[pl.ds(row0 + kk * CH, CH)])
            pltpu.sync_copy(
                rows_v.at[pl.ds(0, 64)], acc.at[pl.ds(row0 + 7 * CH, 64)]
            )

        @pl.when(s == NS - 1)
        def _zero_last():
            for kk in range(8):
                pltpu.sync_copy(rows_v, acc.at[pl.ds(9360 + kk * CH, CH)])

        plsc.subcore_barrier()

        pltpu.sync_copy(n_hbm.at[pl.ds(w * EPW, EPW)], norm_all_v)

        def refill(cix):
            soff = w * EPW + cix * CH
            pltpu.sync_copy(gidx_hbm.at[pl.ds(soff, SUPCH)], gidx_sb)
            pltpu.sync_copy(dst_hbm.at[pl.ds(soff, SUPCH)], dst_sb)

        def fill_idx(cix, b):
            base = lax.rem(cix, SUP) * CH
            for j in range(CH // L):
                pos = base + j * L
                idx_b[b][pl.ds(j * L, L)] = gidx_sb[pl.ds(pos, L)]
                dsti_b[b][pl.ds(j * L, L)] = dst_sb[pl.ds(pos, L)]

        def fire_gather(b):
            pltpu.async_copy(h_hbm.at[idx_b[b]], rows_b[b], sem_g[b])

        def wait_gather(b):
            pltpu.make_async_copy(h_hbm.at[idx_b[b]], rows_b[b], sem_g[b]).wait()

        def fire_scatter(b):
            pltpu.async_copy(rows_b[b], acc.at[dsti_b[b]], sem_s[b], add=True)

        def wait_scatter(b):
            pltpu.make_async_copy(rows_b[b], acc.at[dsti_b[b]], sem_s[b]).wait()

        def scale(cix, b):
            rv = rows_b[b]

            @plsc.parallel_loop(0, CH // L, unroll=2)
            def gbody(g):
                norm16 = norm_all_v[pl.ds(cix * CH + g * L, L)]
                for e in range(L):
                    nb = jnp.take(norm16, jnp.full((L,), e, jnp.int32))
                    for j in range(D // L):
                        rv[g * L + e, pl.ds(j * L, L)] = (
                            rv[g * L + e, pl.ds(j * L, L)] * nb
                        )

        refill(0)
        for b in range(NB):
            fill_idx(b, b)
            fire_gather(b)

        def body(i, _):
            for b in range(NB):
                cix = i * NB + b
                bp = (b - 1) % NB

                @pl.when(cix < NCHUNK)
                def _(cix=cix, b=b, bp=bp):
                    wait_gather(b)
                    scale(cix, b)
                    fire_scatter(b)
                    p = cix - 1

                    @pl.when((cix >= 1) & (p + NB < NCHUNK))
                    def _():
                        wait_scatter(bp)

                        @pl.when(lax.rem(p + NB, SUP) == 0)
                        def _():
                            refill(p + NB)

                        fill_idx(p + NB, bp)
                        fire_gather(bp)

            return 0

        lax.fori_loop(0, (NCHUNK + NB - 1) // NB, body, 0)
        for b in range(NB):
            wait_scatter(b)
        plsc.subcore_barrier()

        @pl.when(s < NS - 1)
        def _out_main():
            for kk in range(7):
                pltpu.sync_copy(acc.at[pl.ds(row0 + kk * CH, CH)], rows_v)
                pltpu.sync_copy(rows_v, out_hbm.at[c, pl.ds(row0 + kk * CH, CH)])
            pltpu.sync_copy(
                acc.at[pl.ds(row0 + 7 * CH, 64)], rows_v.at[pl.ds(0, 64)]
            )
            pltpu.sync_copy(
                rows_v.at[pl.ds(0, 64)], out_hbm.at[c, pl.ds(row0 + 7 * CH, 64)]
            )

        @pl.when(s == NS - 1)
        def _out_last():
            for kk in range(8):
                pltpu.sync_copy(acc.at[pl.ds(9360 + kk * CH, CH)], rows_v)
                pltpu.sync_copy(rows_v, out_hbm.at[c, pl.ds(9360 + kk * CH, CH)])

    return k(h5, gidx, dst, norm_all)


def kernel(thesis_features, professor_features, edge_index, edge_type,
           Wt, bt, gt, bet, Wp, bp, gp, bep,
           Wr1, Wroot1, bc1, g1, be1,
           Wr2, Wroot2, bc2, g2, be2,
           Wr3, Wroot3, bc3, g3, be3):
    src = edge_index[0].astype(jnp.int32)
    dst = edge_index[1].astype(jnp.int32)
    et = edge_type.astype(jnp.int32)

    wall2 = jnp.concatenate([Wr2, Wroot2[None]], axis=0)
    wall3 = jnp.concatenate([Wr3, Wroot3[None]], axis=0)

    x0, h6 = _pre(thesis_features, professor_features,
                  Wt, bt, gt, bet, Wp, bp, gp, bep,
                  jnp.concatenate([Wr1, Wroot1[None]], axis=0))
    deg2 = _deg_sc(dst, et)
    tnorm = _t_table(deg2)
    norm_all, gidx = _prep_sc(src, dst, et, tnorm)

    x = x0
    for bc, g, be, wall_next in (
        (bc1, g1, be1, wall2),
        (bc2, g2, be2, wall3),
        (bc3, g3, be3, None),
    ):
        part = _agg_sc(h6.reshape((R + 1) * N, D), gidx, dst, norm_all)
        if wall_next is None:
            x = _post(part, h6, bc, g, be, x, None)
        else:
            x, h6 = _post(part, h6, bc, g, be, x, wall_next)

    return (x0, x)


# speedup vs baseline: 1.3481x; 1.0499x over previous
import functools

import jax
import jax.numpy as jnp
from jax import lax
from jax.experimental import pallas as pl
from jax.experimental.pallas import tpu as pltpu
from jax.experimental.pallas import tpu_sc as plsc

N_T = 5000
N_P = 5000
N = 10000
E = 320000
D = 128
R = 5

NC = 2
NS = 16
NW = NC * NS
L = 16
EPW = E // NW
CH = 80
NCHUNK = EPW // CH
NR = N * R
NRP = 50176
DPT = NRP // NS
RPT = N // NS
BR = 1000


def _ln(y, g, b):
    m = jnp.mean(y, axis=-1, keepdims=True)
    v = jnp.mean((y - m) ** 2, axis=-1, keepdims=True)
    return (y - m) / jnp.sqrt(v + 1e-5) * g + b


def _pre_body(x_ref, w_ref, b_ref, g_ref, be_ref, wall_ref, o_ref, h_ref):
    y = jnp.dot(x_ref[0, 0], w_ref[0], preferred_element_type=jnp.float32)
    y = _ln(y + b_ref[0], g_ref[0], be_ref[0])
    y = jnp.maximum(y, 0.0)
    o_ref[...] = y
    for r in range(R + 1):
        h_ref[r] = jnp.dot(y, wall_ref[r], preferred_element_type=jnp.float32)


def _pre(tf, pf, Wt, bt, gt, bet, Wp, bp, gp, bep, wall1):
    nb = N_T // BR
    xs = jnp.stack([tf, pf]).reshape(2, nb, BR, D)
    ws = jnp.stack([Wt, Wp])
    bs = jnp.stack([bt, bp]).reshape(2, 1, D)
    gs = jnp.stack([gt, gp]).reshape(2, 1, D)
    bes = jnp.stack([bet, bep]).reshape(2, 1, D)
    return pl.pallas_call(
        _pre_body,
        grid=(N // BR,),
        in_specs=[
            pl.BlockSpec((1, 1, BR, D), lambda i: (i // nb, i % nb, 0, 0)),
            pl.BlockSpec((1, D, D), lambda i: (i // nb, 0, 0)),
            pl.BlockSpec((1, 1, D), lambda i: (i // nb, 0, 0)),
            pl.BlockSpec((1, 1, D), lambda i: (i // nb, 0, 0)),
            pl.BlockSpec((1, 1, D), lambda i: (i // nb, 0, 0)),
            pl.BlockSpec((R + 1, D, D), lambda i: (0, 0, 0)),
        ],
        out_specs=[
            pl.BlockSpec((BR, D), lambda i: (i, 0)),
            pl.BlockSpec((R + 1, BR, D), lambda i: (0, i, 0)),
        ],
        out_shape=[
            jax.ShapeDtypeStruct((N, D), jnp.float32),
            jax.ShapeDtypeStruct((R + 1, N, D), jnp.float32),
        ],
    )(xs, ws, bs, gs, bes, wall1)


def _post_last_body(part_ref, h_ref, bc_ref, g_ref, be_ref, res_ref, o_ref):
    y = part_ref[0] + part_ref[1] + h_ref[0] + bc_ref[...]
    y = _ln(y, g_ref[...], be_ref[...])
    o_ref[...] = y + res_ref[...]


def _post_next_body(part_ref, h_ref, bc_ref, g_ref, be_ref, res_ref, wall_ref,
                    o_ref, hn_ref):
    y = part_ref[0] + part_ref[1] + h_ref[0] + bc_ref[...]
    y = _ln(y, g_ref[...], be_ref[...])
    y = jnp.maximum(y, 0.0) + res_ref[...]
    o_ref[...] = y
    for r in range(R + 1):
        hn_ref[r] = jnp.dot(y, wall_ref[r], preferred_element_type=jnp.float32)


def _post(part, h6, bc, g, be, res, wall_next):
    base_specs = [
        pl.BlockSpec((NC, BR, D), lambda i: (0, i, 0)),
        pl.BlockSpec((1, BR, D), lambda i: (R, i, 0)),
        pl.BlockSpec((1, D), lambda i: (0, 0)),
        pl.BlockSpec((1, D), lambda i: (0, 0)),
        pl.BlockSpec((1, D), lambda i: (0, 0)),
        pl.BlockSpec((BR, D), lambda i: (i, 0)),
    ]
    args = (part, h6, bc.reshape(1, D), g.reshape(1, D), be.reshape(1, D), res)
    if wall_next is None:
        return pl.pallas_call(
            _post_last_body,
            grid=(N // BR,),
            in_specs=base_specs,
            out_specs=pl.BlockSpec((BR, D), lambda i: (i, 0)),
            out_shape=jax.ShapeDtypeStruct((N, D), jnp.float32),
        )(*args)
    return pl.pallas_call(
        _post_next_body,
        grid=(N // BR,),
        in_specs=base_specs + [pl.BlockSpec((R + 1, D, D), lambda i: (0, 0, 0))],
        out_specs=[
            pl.BlockSpec((BR, D), lambda i: (i, 0)),
            pl.BlockSpec((R + 1, BR, D), lambda i: (0, i, 0)),
        ],
        out_shape=[
            jax.ShapeDtypeStruct((N, D), jnp.float32),
            jax.ShapeDtypeStruct((R + 1, N, D), jnp.float32),
        ],
    )(*args, wall_next)


def _t_body(d_ref, o_ref):
    o_ref[...] = 1.0 / jnp.maximum(d_ref[0] + d_ref[1], 1.0)


def _t_table(deg2):
    out = pl.pallas_call(
        _t_body,
        in_specs=[pl.BlockSpec((NC, NRP // D, D), lambda: (0, 0, 0))],
        out_specs=pl.BlockSpec((NRP // D, D), lambda: (0, 0)),
        out_shape=jax.ShapeDtypeStruct((NRP // D, D), jnp.float32),
    )(deg2.reshape(NC, NRP // D, D))
    return out.reshape(NRP)


def _sc_mesh():
    return plsc.VectorSubcoreMesh(core_axis_name="c", subcore_axis_name="s")


def _deg_sc(dst, et):
    @functools.partial(
        pl.kernel,
        out_type=jax.ShapeDtypeStruct((NC * NRP,), jnp.float32),
        mesh=_sc_mesh(),
        scratch_types=[
            pltpu.VMEM((EPW,), jnp.int32),
            pltpu.VMEM((EPW,), jnp.int32),
            pltpu.VMEM((CH,), jnp.int32),
            pltpu.VMEM((CH,), jnp.float32),
            pltpu.VMEM((DPT,), jnp.float32),
            pltpu.VMEM_SHARED((NRP,), jnp.float32),
        ],
    )
    def k(dst_hbm, et_hbm, out_hbm, dst_all, et_all, key_v, ones_v, zb_v, acc):
        c = lax.axis_index("c")
        s = lax.axis_index("s")
        w = c * NS + s
        one16 = jnp.ones((L,), jnp.float32)
        z16 = jnp.zeros((L,), jnp.float32)

        def fill(i, _):
            ones_v[pl.ds(i * L, L)] = one16
            return 0

        lax.fori_loop(0, CH // L, fill, 0)

        def fillz(i, _):
            zb_v[pl.ds(i * L, L)] = z16
            return 0

        lax.fori_loop(0, DPT // L, fillz, 0)
        pltpu.sync_copy(zb_v, acc.at[pl.ds(s * DPT, DPT)])
        plsc.subcore_barrier()

        pltpu.sync_copy(dst_hbm.at[pl.ds(w * EPW, EPW)], dst_all)
        pltpu.sync_copy(et_hbm.at[pl.ds(w * EPW, EPW)], et_all)

        def chunk(cix, _):
            base = cix * CH
            for j in range(CH // L):
                pos = base + j * L
                d16 = dst_all[pl.ds(pos, L)]
                e16 = et_all[pl.ds(pos, L)]
                key_v[pl.ds(j * L, L)] = d16 * R + e16
            pltpu.sync_copy(ones_v, acc.at[key_v], add=True)
            return 0

        lax.fori_loop(0, NCHUNK, chunk, 0)
        plsc.subcore_barrier()
        pltpu.sync_copy(acc.at[pl.ds(s * DPT, DPT)], zb_v)
        pltpu.sync_copy(zb_v, out_hbm.at[pl.ds(c * NRP + s * DPT, DPT)])

    return k(dst, et).reshape(NC, NRP)


def _prep_sc(src, dst, et, tnorm):
    @functools.partial(
        pl.kernel,
        out_type=(
            jax.ShapeDtypeStruct((E,), jnp.float32),
            jax.ShapeDtypeStruct((E,), jnp.int32),
        ),
        mesh=_sc_mesh(),
        scratch_types=[
            pltpu.VMEM((EPW,), jnp.int32),
            pltpu.VMEM((EPW,), jnp.int32),
            pltpu.VMEM((EPW,), jnp.int32),
            pltpu.VMEM((EPW,), jnp.float32),
            pltpu.VMEM((EPW,), jnp.int32),
            [pltpu.VMEM((CH,), jnp.int32)] * 4,
            [pltpu.SemaphoreType.DMA] * 4,
        ],
    )
    def k(src_hbm, dst_hbm, et_hbm, t_hbm, nout_hbm, gout_hbm,
          src_all, dst_all, et_all, norm_slab, gidx_slab, keys, sems):
        c = lax.axis_index("c")
        s = lax.axis_index("s")
        w = c * NS + s
        pltpu.sync_copy(src_hbm.at[pl.ds(w * EPW, EPW)], src_all)
        pltpu.sync_copy(dst_hbm.at[pl.ds(w * EPW, EPW)], dst_all)
        pltpu.sync_copy(et_hbm.at[pl.ds(w * EPW, EPW)], et_all)

        def gfill(i, _):
            pos = i * L
            gidx_slab[pl.ds(pos, L)] = (
                et_all[pl.ds(pos, L)] * N + src_all[pl.ds(pos, L)]
            )
            return 0

        lax.fori_loop(0, EPW // L, gfill, 0)
        pltpu.sync_copy(gidx_slab, gout_hbm.at[pl.ds(w * EPW, EPW)])

        def fill_keys(cix, kv):
            base = cix * CH
            for j in range(CH // L):
                pos = base + j * L
                kv[pl.ds(j * L, L)] = (
                    dst_all[pl.ds(pos, L)] * R + et_all[pl.ds(pos, L)]
                )

        def fire(cix, b):
            pltpu.async_copy(
                t_hbm.at[keys[b]], norm_slab.at[pl.ds(cix * CH, CH)], sems[b]
            )

        def drain(cix, b):
            pltpu.make_async_copy(
                t_hbm.at[keys[b]], norm_slab.at[pl.ds(cix * CH, CH)], sems[b]
            ).wait()

        NBK = 4
        for q in range(NBK):
            fill_keys(q, keys[q])
            fire(q, q)

        def body(i, _):
            for q in range(NBK):
                cix = i * NBK + q

                @pl.when(cix < NCHUNK)
                def _(cix=cix, q=q):
                    drain(cix, q)

                    @pl.when(cix + NBK < NCHUNK)
                    def _():
                        fill_keys(cix + NBK, keys[q])
                        fire(cix + NBK, q)

            return 0

        lax.fori_loop(0, (NCHUNK + NBK - 1) // NBK, body, 0)
        pltpu.sync_copy(norm_slab, nout_hbm.at[pl.ds(w * EPW, EPW)])

    return k(src, dst, et, tnorm)


NB = 3
SUP = 25
SUPCH = SUP * CH


def _agg_sc(h5, gidx, dst, norm_all):
    @functools.partial(
        pl.kernel,
        out_type=jax.ShapeDtypeStruct((NC, N, D), jnp.float32),
        mesh=_sc_mesh(),
        scratch_types=[
            pltpu.VMEM((SUPCH,), jnp.int32),
            pltpu.VMEM((SUPCH,), jnp.int32),
            pltpu.VMEM((EPW,), jnp.float32),
            [pltpu.VMEM((CH,), jnp.int32)] * NB,
            [pltpu.VMEM((CH,), jnp.int32)] * NB,
            [pltpu.VMEM((CH, D), jnp.float32)] * NB,
            [pltpu.SemaphoreType.DMA] * NB,
            [pltpu.SemaphoreType.DMA] * NB,
            pltpu.VMEM_SHARED((N, D), jnp.float32),
        ],
    )
    def k(h_hbm, gidx_hbm, dst_hbm, n_hbm, out_hbm,
          gidx_sb, dst_sb, norm_all_v, idx_b, dsti_b, rows_b,
          sem_g, sem_s, acc):
        c = lax.axis_index("c")
        s = lax.axis_index("s")
        w = c * NS + s
        z16 = jnp.zeros((L,), jnp.float32)
        rows_v = rows_b[0]

        def zr(i, _):
            for j in range(D // L):
                rows_v[i, pl.ds(j * L, L)] = z16
            return 0

        lax.fori_loop(0, CH, zr, 0)
        row0 = pl.multiple_of(s * 624, 8)

        @pl.when(s < NS - 1)
        def _zero_main():
            for kk in range(7):
                pltpu.async_copy(
                    rows_v, acc.at[pl.ds(row0 + kk * CH, CH)], sem_s[0]
                )
            pltpu.async_copy(
                rows_v.at[pl.ds(0, 64)], acc.at[pl.ds(row0 + 7 * CH, 64)],
                sem_s[1],
            )
            for kk in range(7):
                pltpu.make_async_copy(
                    rows_v, acc.at[pl.ds(row0 + kk * CH, CH)], sem_s[0]
                ).wait()
            pltpu.make_async_copy(
                rows_v.at[pl.ds(0, 64)], acc.at[pl.ds(row0 + 7 * CH, 64)],
                sem_s[1],
            ).wait()

        @pl.when(s == NS - 1)
        def _zero_last():
            for kk in range(8):
                pltpu.async_copy(
                    rows_v, acc.at[pl.ds(9360 + kk * CH, CH)], sem_s[0]
                )
            for kk in range(8):
                pltpu.make_async_copy(
                    rows_v, acc.at[pl.ds(9360 + kk * CH, CH)], sem_s[0]
                ).wait()

        plsc.subcore_barrier()

        pltpu.sync_copy(n_hbm.at[pl.ds(w * EPW, EPW)], norm_all_v)

        def refill(cix):
            soff = w * EPW + cix * CH
            pltpu.sync_copy(gidx_hbm.at[pl.ds(soff, SUPCH)], gidx_sb)
            pltpu.sync_copy(dst_hbm.at[pl.ds(soff, SUPCH)], dst_sb)

        def fill_idx(cix, b):
            base = lax.rem(cix, SUP) * CH
            for j in range(CH // L):
                pos = base + j * L
                idx_b[b][pl.ds(j * L, L)] = gidx_sb[pl.ds(pos, L)]
                dsti_b[b][pl.ds(j * L, L)] = dst_sb[pl.ds(pos, L)]

        def fire_gather(b):
            pltpu.async_copy(h_hbm.at[idx_b[b]], rows_b[b], sem_g[b])

        def wait_gather(b):
            pltpu.make_async_copy(h_hbm.at[idx_b[b]], rows_b[b], sem_g[b]).wait()

        def fire_scatter(b):
            pltpu.async_copy(rows_b[b], acc.at[dsti_b[b]], sem_s[b], add=True)

        def wait_scatter(b):
            pltpu.make_async_copy(rows_b[b], acc.at[dsti_b[b]], sem_s[b]).wait()

        def scale(cix, b):
            rv = rows_b[b]

            @plsc.parallel_loop(0, CH // L, unroll=2)
            def gbody(g):
                norm16 = norm_all_v[pl.ds(cix * CH + g * L, L)]
                for e in range(L):
                    nb = jnp.take(norm16, jnp.full((L,), e, jnp.int32))
                    for j in range(D // L):
                        rv[g * L + e, pl.ds(j * L, L)] = (
                            rv[g * L + e, pl.ds(j * L, L)] * nb
                        )

        refill(0)
        for b in range(NB):
            fill_idx(b, b)
            fire_gather(b)

        def body(i, _):
            for b in range(NB):
                cix = i * NB + b
                bp = (b - 1) % NB

                @pl.when(cix < NCHUNK)
                def _(cix=cix, b=b, bp=bp):
                    wait_gather(b)
                    scale(cix, b)
                    fire_scatter(b)
                    p = cix - 1

                    @pl.when((cix >= 1) & (p + NB < NCHUNK))
                    def _():
                        wait_scatter(bp)

                        @pl.when(lax.rem(p + NB, SUP) == 0)
                        def _():
                            refill(p + NB)

                        fill_idx(p + NB, bp)
                        fire_gather(bp)

            return 0

        lax.fori_loop(0, (NCHUNK + NB - 1) // NB, body, 0)
        for b in range(NB):
            wait_scatter(b)
        plsc.subcore_barrier()

        def _readout(base, nfull, tail64):
            for kk in range(nfull):
                b = kk % NB
                if kk >= NB:
                    pltpu.make_async_copy(
                        rows_b[b],
                        out_hbm.at[c, pl.ds(base + (kk - NB) * CH, CH)],
                        sem_g[b],
                    ).wait()
                pltpu.sync_copy(acc.at[pl.ds(base + kk * CH, CH)], rows_b[b])
                pltpu.async_copy(
                    rows_b[b], out_hbm.at[c, pl.ds(base + kk * CH, CH)], sem_g[b]
                )
            for kk in range(max(0, nfull - NB), nfull):
                pltpu.make_async_copy(
                    rows_b[kk % NB],
                    out_hbm.at[c, pl.ds(base + kk * CH, CH)],
                    sem_g[kk % NB],
                ).wait()
            if tail64:
                pltpu.sync_copy(
                    acc.at[pl.ds(base + nfull * CH, 64)],
                    rows_b[0].at[pl.ds(0, 64)],
                )
                pltpu.sync_copy(
                    rows_b[0].at[pl.ds(0, 64)],
                    out_hbm.at[c, pl.ds(base + nfull * CH, 64)],
                )

        @pl.when(s < NS - 1)
        def _out_main():
            _readout(row0, 7, True)

        @pl.when(s == NS - 1)
        def _out_last():
            _readout(9360, 8, False)

    return k(h5, gidx, dst, norm_all)


def kernel(thesis_features, professor_features, edge_index, edge_type,
           Wt, bt, gt, bet, Wp, bp, gp, bep,
           Wr1, Wroot1, bc1, g1, be1,
           Wr2, Wroot2, bc2, g2, be2,
           Wr3, Wroot3, bc3, g3, be3):
    src = edge_index[0].astype(jnp.int32)
    dst = edge_index[1].astype(jnp.int32)
    et = edge_type.astype(jnp.int32)

    wall2 = jnp.concatenate([Wr2, Wroot2[None]], axis=0)
    wall3 = jnp.concatenate([Wr3, Wroot3[None]], axis=0)

    x0, h6 = _pre(thesis_features, professor_features,
                  Wt, bt, gt, bet, Wp, bp, gp, bep,
                  jnp.concatenate([Wr1, Wroot1[None]], axis=0))
    deg2 = _deg_sc(dst, et)
    tnorm = _t_table(deg2)
    norm_all, gidx = _prep_sc(src, dst, et, tnorm)

    x = x0
    for bc, g, be, wall_next in (
        (bc1, g1, be1, wall2),
        (bc2, g2, be2, wall3),
        (bc3, g3, be3, None),
    ):
        part = _agg_sc(h6.reshape((R + 1) * N, D), gidx, dst, norm_all)
        if wall_next is None:
            x = _post(part, h6, bc, g, be, x, None)
        else:
            x, h6 = _post(part, h6, bc, g, be, x, wall_next)

    return (x0, x)
